# Initial kernel scaffold; baseline (speedup 1.0000x reference)
import jax
import jax.numpy as jnp
from jax.experimental import pallas as pl


def kernel(visit_order, pos_embed_weight):
    raise NotImplementedError("write your pallas kernel here")


# speedup vs baseline: 4.1105x; 4.1105x over previous
import functools

import jax
import jax.numpy as jnp
from jax import lax
from jax.experimental import pallas as pl
from jax.experimental.pallas import tpu as pltpu
from jax.experimental.pallas import tpu_sc as plsc

EMBED_NUM = 1000
EMBED_DIM = 64
BATCH = 16384
HIST = 200

_B = BATCH * HIST
_CH = 128
_NCHUNK = _B // _CH
_GK = 8
_NC = 2
_NS = 16
_NW = _NC * _NS
_CPW = _NCHUNK // _NW
_NGROUP = _CPW // _GK


def _sc_gather(idx2d, table):
    mesh = plsc.VectorSubcoreMesh(core_axis_name="c", subcore_axis_name="s")

    @functools.partial(
        pl.kernel,
        mesh=mesh,
        compiler_params=pltpu.CompilerParams(use_tc_tiling_on_sc=False),
        out_type=jax.ShapeDtypeStruct((_B, EMBED_DIM), jnp.float32),
        scratch_types=[
            pltpu.VMEM((_GK, _CH), jnp.int32),
            pltpu.VMEM((_GK, _CH, EMBED_DIM), jnp.float32),
            pltpu.SemaphoreType.DMA,
        ],
    )
    def k(idx_hbm, table_hbm, out_hbm, idx_v, rows_v, sem):
        wid = lax.axis_index("s") * _NC + lax.axis_index("c")
        chunk0 = wid * _CPW

        def group(g, carry):
            row0 = chunk0 + g * _GK
            pltpu.sync_copy(idx_hbm.at[pl.ds(row0, _GK)], idx_v)
            copies = []
            for j in range(_GK):
                copies.append(
                    pltpu.async_copy(table_hbm.at[idx_v.at[j]], rows_v.at[j], sem)
                )
            for j in range(_GK):
                copies[j].wait()
                pltpu.sync_copy(
                    rows_v.at[j], out_hbm.at[pl.ds((row0 + j) * _CH, _CH)]
                )
            return carry

        lax.fori_loop(0, _NGROUP, group, 0)

    return k(idx2d, table)


def kernel(visit_order, pos_embed_weight):
    idx2d = visit_order.astype(jnp.int32).reshape(_NCHUNK, _CH)
    flat = _sc_gather(idx2d, pos_embed_weight)
    return flat.reshape(BATCH, HIST, EMBED_DIM)


# speedup vs baseline: 4.1268x; 1.0040x over previous
import functools

import jax
import jax.numpy as jnp
from jax import lax
from jax.experimental import pallas as pl
from jax.experimental.pallas import tpu as pltpu
from jax.experimental.pallas import tpu_sc as plsc

EMBED_NUM = 1000
EMBED_DIM = 64
BATCH = 16384
HIST = 200

_B = BATCH * HIST
_CH = 128
_NCHUNK = _B // _CH
_GI = 5
_NC = 2
_NS = 16
_NW = _NC * _NS
_NGW = _NCHUNK // (_NW * _GI)
_NI = _NGW // 2


def _sc_gather(idx3d, table):
    mesh = plsc.VectorSubcoreMesh(core_axis_name="c", subcore_axis_name="s")

    @functools.partial(
        pl.kernel,
        mesh=mesh,
        compiler_params=pltpu.CompilerParams(use_tc_tiling_on_sc=False),
        out_type=jax.ShapeDtypeStruct((_NCHUNK, _CH, EMBED_DIM), jnp.float32),
        scratch_types=[
            pltpu.VMEM((2, _GI, _CH), jnp.int32),
            pltpu.VMEM((2, _GI, _CH, EMBED_DIM), jnp.float32),
            pltpu.SemaphoreType.DMA,
            pltpu.SemaphoreType.DMA,
            pltpu.SemaphoreType.DMA,
            pltpu.SemaphoreType.DMA,
            pltpu.SemaphoreType.DMA,
            pltpu.SemaphoreType.DMA,
        ],
    )
    def k(idx_hbm, table_hbm, out_hbm, idx_v, rows_v, sg0, sg1, sw0, sw1, si0, si1):
        wid = lax.axis_index("s") * _NC + lax.axis_index("c")
        gbase = wid * _NGW
        sg = (sg0, sg1)
        sw = (sw0, sw1)
        si = (si0, si1)

        def chunk0(g):
            return (gbase + g) * _GI

        def fire_gathers(g, b):
            for j in range(_GI):
                pltpu.async_copy(
                    table_hbm.at[idx_v.at[b].at[j]], rows_v.at[b].at[j], sg[b]
                )

        def drain_gathers(b):
            pltpu.make_async_copy(out_hbm.at[pl.ds(0, _GI)], rows_v.at[b], sg[b]).wait()

        def fire_write(g, b):
            pltpu.async_copy(rows_v.at[b], out_hbm.at[pl.ds(chunk0(g), _GI)], sw[b])

        def drain_write(b):
            pltpu.make_async_copy(out_hbm.at[pl.ds(0, _GI)], rows_v.at[b], sw[b]).wait()

        def fire_idx(g, b):
            pltpu.async_copy(idx_hbm.at[pl.ds(chunk0(g), _GI)], idx_v.at[b], si[b])

        def drain_idx(b):
            pltpu.make_async_copy(idx_hbm.at[pl.ds(0, _GI)], idx_v.at[b], si[b]).wait()

        pltpu.sync_copy(idx_hbm.at[pl.ds(chunk0(0), _GI)], idx_v.at[0])

        def body(i, carry):
            ga = 2 * i
            gb = 2 * i + 1

            @pl.when(i >= 1)
            def _():
                drain_write(0)
                drain_idx(0)

            fire_gathers(ga, 0)

            @pl.when(i >= 1)
            def _():
                drain_gathers(1)
                fire_write(gb - 2, 1)

            fire_idx(gb, 1)

            @pl.when(i >= 1)
            def _():
                drain_write(1)

            drain_idx(1)
            fire_gathers(gb, 1)
            drain_gathers(0)
            fire_write(ga, 0)

            @pl.when(i < _NI - 1)
            def _():
                fire_idx(ga + 2, 0)

            return carry

        lax.fori_loop(0, _NI, body, 0)

        drain_gathers(1)
        fire_write(_NGW - 1, 1)
        drain_write(0)
        drain_write(1)

    return k(idx3d, table)


def kernel(visit_order, pos_embed_weight):
    idx2d = visit_order.astype(jnp.int32).reshape(_NCHUNK, _CH)
    flat = _sc_gather(idx2d, pos_embed_weight)
    return flat.reshape(BATCH, HIST, EMBED_DIM)
